# Initial kernel scaffold; baseline (speedup 1.0000x reference)
import jax
import jax.numpy as jnp
from jax.experimental import pallas as pl


def kernel(h, ei, ea, batch, W1, b1, W2, b2):
    raise NotImplementedError("write your pallas kernel here")


# speedup vs baseline: 7.2663x; 7.2663x over previous
import functools

import jax
import jax.numpy as jnp
from jax import lax
from jax.experimental import pallas as pl
from jax.experimental.pallas import tpu as pltpu
from jax.experimental.pallas import tpu_sc as plsc

_N_NODES = 10000
_N_EDGES = 320000
_D = 128
_E_DIM = 16
_G = 64

_NC = 2
_NS = 16
_NW = _NC * _NS
_EPW = _N_EDGES // _NW
_CH = 400
_NCH = _EPW // _CH

_EBLK = 8000


def _prep_body(h_ref, batch_ref, w1_ref, b1_ref, a_ref, b_ref):
    h = h_ref[...]
    batch = batch_ref[0, 0, :]
    ids = lax.broadcasted_iota(jnp.int32, (_G, _N_NODES), 0)
    oh = jnp.where(ids == batch[None, :], 1.0, 0.0)
    sums = jnp.dot(oh, h, preferred_element_type=jnp.float32)
    counts = jnp.sum(oh, axis=1, keepdims=True)
    means = sums / jnp.maximum(counts, 1.0)
    mp = jnp.dot(means, w1_ref[272:400, :], preferred_element_type=jnp.float32)
    ids2 = lax.broadcasted_iota(jnp.int32, (_N_NODES, _G), 1)
    oh2 = jnp.where(batch[:, None] == ids2, 1.0, 0.0)
    a_ref[...] = (jnp.dot(h, w1_ref[0:128, :], preferred_element_type=jnp.float32)
                  + jnp.dot(oh2, mp, preferred_element_type=jnp.float32)
                  + b1_ref[...])
    b_ref[...] = jnp.dot(h, w1_ref[128:256, :], preferred_element_type=jnp.float32)


def _mlp_body(g_ref, ea_ref, w1e_ref, w2_ref, b2_ref, out_ref):
    z = g_ref[...] + jnp.dot(ea_ref[...], w1e_ref[...], preferred_element_type=jnp.float32)
    out_ref[...] = (jnp.dot(jnp.maximum(z, 0.0), w2_ref[...],
                            preferred_element_type=jnp.float32) + b2_ref[...])


def _gather_sum(a_tab, b_tab, s_idx, t_idx):
    mesh = plsc.VectorSubcoreMesh(core_axis_name="c", subcore_axis_name="s")

    @functools.partial(
        pl.kernel, mesh=mesh,
        out_type=jax.ShapeDtypeStruct((_N_EDGES, _D), jnp.float32),
        scratch_types=[
            pltpu.VMEM((_CH,), jnp.int32),
            pltpu.VMEM((_CH,), jnp.int32),
            pltpu.VMEM((_CH, _D), jnp.float32),
            pltpu.VMEM((_CH, _D), jnp.float32),
            pltpu.SemaphoreType.DMA,
            pltpu.SemaphoreType.DMA,
        ],
    )
    def k(a_hbm, b_hbm, s_hbm, t_hbm, out_hbm, sv, tv, ra, rb, sem_a, sem_b):
        wid = lax.axis_index("s") * _NC + lax.axis_index("c")
        base = wid * _EPW

        def chunk(j, carry):
            off = base + j * _CH
            pltpu.sync_copy(s_hbm.at[pl.ds(off, _CH)], sv)
            pltpu.sync_copy(t_hbm.at[pl.ds(off, _CH)], tv)
            ca = pltpu.async_copy(a_hbm.at[sv], ra, sem_a)
            cb = pltpu.async_copy(b_hbm.at[tv], rb, sem_b)
            ca.wait()
            cb.wait()

            def row(r, c2):
                for cc in range(_D // 16):
                    sl = pl.ds(cc * 16, 16)
                    plsc.addupdate(ra.at[r, sl], rb[r, sl])
                return c2

            lax.fori_loop(0, _CH, row, 0)
            pltpu.sync_copy(ra, out_hbm.at[pl.ds(off, _CH)])
            return carry

        lax.fori_loop(0, _NCH, chunk, 0)

    return k(a_tab, b_tab, s_idx, t_idx)


def kernel(h, ei, ea, batch, W1, b1, W2, b2):
    s = ei[0].astype(jnp.int32)
    t = ei[1].astype(jnp.int32)
    batch3 = batch.astype(jnp.int32).reshape(1, 1, _N_NODES)
    a_tab, b_tab = pl.pallas_call(
        _prep_body,
        out_shape=(jax.ShapeDtypeStruct((_N_NODES, _D), jnp.float32),
                   jax.ShapeDtypeStruct((_N_NODES, _D), jnp.float32)),
    )(h, batch3, W1, b1.reshape(1, _D))
    g = _gather_sum(a_tab, b_tab, s, t)
    out = pl.pallas_call(
        _mlp_body,
        grid=(_N_EDGES // _EBLK,),
        in_specs=[
            pl.BlockSpec((_EBLK, _D), lambda i: (i, 0)),
            pl.BlockSpec((_EBLK, _E_DIM), lambda i: (i, 0)),
            pl.BlockSpec((_E_DIM, _D), lambda i: (0, 0)),
            pl.BlockSpec((_D, 1), lambda i: (0, 0)),
            pl.BlockSpec((1, 1), lambda i: (0, 0)),
        ],
        out_specs=pl.BlockSpec((_EBLK, 1), lambda i: (i, 0)),
        out_shape=jax.ShapeDtypeStruct((_N_EDGES, 1), jnp.float32),
    )(g, ea, W1[256:272], W2, b2.reshape(1, 1))
    return out


# speedup vs baseline: 9.4365x; 1.2987x over previous
import functools

import jax
import jax.numpy as jnp
from jax import lax
from jax.experimental import pallas as pl
from jax.experimental.pallas import tpu as pltpu
from jax.experimental.pallas import tpu_sc as plsc

_N_NODES = 10000
_N_EDGES = 320000
_D = 128
_E_DIM = 16
_G = 64

_NC = 2
_NS = 16
_NW = _NC * _NS
_EPW = _N_EDGES // _NW
_CH = 400
_NCH = _EPW // _CH

_EBLK = 6400


def _prep_body(h_ref, batch_ref, w1_ref, b1_ref, a_ref, b_ref):
    h = h_ref[...]
    batch = batch_ref[0, 0, :]
    ids = lax.broadcasted_iota(jnp.int32, (_G, _N_NODES), 0)
    oh = jnp.where(ids == batch[None, :], 1.0, 0.0)
    sums = jnp.dot(oh, h, preferred_element_type=jnp.float32)
    counts = jnp.sum(oh, axis=1, keepdims=True)
    means = sums / jnp.maximum(counts, 1.0)
    mp = jnp.dot(means, w1_ref[272:400, :], preferred_element_type=jnp.float32)
    ids2 = lax.broadcasted_iota(jnp.int32, (_N_NODES, _G), 1)
    oh2 = jnp.where(batch[:, None] == ids2, 1.0, 0.0)
    a_ref[...] = (jnp.dot(h, w1_ref[0:128, :], preferred_element_type=jnp.float32)
                  + jnp.dot(oh2, mp, preferred_element_type=jnp.float32)
                  + b1_ref[...])
    b_ref[...] = jnp.dot(h, w1_ref[128:256, :], preferred_element_type=jnp.float32)


def _mlp_body(g_ref, ea3_ref, w1e_ref, w2_ref, b2_ref, out_ref):
    ea = ea3_ref[...].reshape(_EBLK, _E_DIM)
    z = g_ref[...] + jnp.dot(ea, w1e_ref[...], preferred_element_type=jnp.float32)
    rz3 = jnp.maximum(z, 0.0).reshape(_EBLK // 128, 128, _D)
    d = lax.dot_general(w2_ref[...], rz3, (((0,), (2,)), ((), ())),
                        preferred_element_type=jnp.float32)
    out_ref[...] = d + b2_ref[...]


def _gather_sum(a_tab, b_tab, s_idx, t_idx):
    mesh = plsc.VectorSubcoreMesh(core_axis_name="c", subcore_axis_name="s")

    @functools.partial(
        pl.kernel, mesh=mesh,
        out_type=jax.ShapeDtypeStruct((_N_EDGES, _D), jnp.float32),
        scratch_types=[
            pltpu.VMEM((_CH,), jnp.int32),
            pltpu.VMEM((_CH,), jnp.int32),
            pltpu.VMEM((_CH, _D), jnp.float32),
            pltpu.VMEM((_CH, _D), jnp.float32),
            pltpu.SemaphoreType.DMA,
            pltpu.SemaphoreType.DMA,
        ],
    )
    def k(a_hbm, b_hbm, s_hbm, t_hbm, out_hbm, sv, tv, ra, rb, sem_a, sem_b):
        wid = lax.axis_index("s") * _NC + lax.axis_index("c")
        base = wid * _EPW

        def chunk(j, carry):
            off = base + j * _CH
            pltpu.sync_copy(s_hbm.at[pl.ds(off, _CH)], sv)
            pltpu.sync_copy(t_hbm.at[pl.ds(off, _CH)], tv)
            ca = pltpu.async_copy(a_hbm.at[sv], ra, sem_a)
            cb = pltpu.async_copy(b_hbm.at[tv], rb, sem_b)
            ca.wait()
            cb.wait()

            def row(r, c2):
                for cc in range(_D // 16):
                    sl = pl.ds(cc * 16, 16)
                    plsc.addupdate(ra.at[r, sl], rb[r, sl])
                return c2

            lax.fori_loop(0, _CH, row, 0)
            pltpu.sync_copy(ra, out_hbm.at[pl.ds(off, _CH)])
            return carry

        lax.fori_loop(0, _NCH, chunk, 0)

    return k(a_tab, b_tab, s_idx, t_idx)


def kernel(h, ei, ea, batch, W1, b1, W2, b2):
    s = ei[0].astype(jnp.int32)
    t = ei[1].astype(jnp.int32)
    batch3 = batch.astype(jnp.int32).reshape(1, 1, _N_NODES)
    a_tab, b_tab = pl.pallas_call(
        _prep_body,
        out_shape=(jax.ShapeDtypeStruct((_N_NODES, _D), jnp.float32),
                   jax.ShapeDtypeStruct((_N_NODES, _D), jnp.float32)),
    )(h, batch3, W1, b1.reshape(1, _D))
    g = _gather_sum(a_tab, b_tab, s, t)
    ea3 = ea.reshape(_N_EDGES // 128, 128, _E_DIM)
    out = pl.pallas_call(
        _mlp_body,
        grid=(_N_EDGES // _EBLK,),
        in_specs=[
            pl.BlockSpec((_EBLK, _D), lambda i: (i, 0)),
            pl.BlockSpec((_EBLK // 128, 128, _E_DIM), lambda i: (i, 0, 0)),
            pl.BlockSpec((_E_DIM, _D), lambda i: (0, 0)),
            pl.BlockSpec((_D, 1), lambda i: (0, 0)),
            pl.BlockSpec((1, 1), lambda i: (0, 0)),
        ],
        out_specs=pl.BlockSpec((1, _EBLK // 128, 128), lambda i: (i, 0, 0)),
        out_shape=jax.ShapeDtypeStruct(
            (_N_EDGES // _EBLK, _EBLK // 128, 128), jnp.float32),
    )(g, ea3, W1[256:272], W2, b2.reshape(1, 1))
    return out.reshape(_N_EDGES, 1)
